# Initial kernel scaffold; baseline (speedup 1.0000x reference)
import jax
import jax.numpy as jnp
from jax.experimental import pallas as pl


def kernel(h_tr, h_re, codebook_tr, codebook_re):
    raise NotImplementedError("write your pallas kernel here")


# speedup vs baseline: 8.6852x; 8.6852x over previous
import jax
import jax.numpy as jnp
from jax.experimental import pallas as pl
from jax.experimental.pallas import tpu as pltpu

_BETA = 0.25


def _vq_body(h_ref, cb_ref, q_ref, g_ref, idx_ref, ss_ref, *, rb, k):
    h = h_ref[...]
    cb = cb_ref[...]
    h_sq = jnp.sum(h * h, axis=1, keepdims=True)
    c_sq = jnp.sum(cb * cb, axis=1).reshape(1, k)
    mm = jax.lax.dot_general(h, cb, (((1,), (1,)), ((), ())),
                             preferred_element_type=jnp.float32)
    dist = h_sq + c_sq - 2.0 * mm
    idx = jnp.argmin(dist, axis=1)
    iota = jax.lax.broadcasted_iota(jnp.int32, (rb, k), 1)
    q = (iota == idx[:, None]).astype(jnp.float32)
    q_ref[...] = q
    g = jax.lax.dot_general(q, cb, (((1,), (0,)), ((), ())),
                            preferred_element_type=jnp.float32)
    g_ref[...] = g
    idx_ref[...] = idx[:, None]
    diff = h - g
    part = jnp.sum(diff * diff).reshape(1, 1)
    prev = ss_ref[...]
    ss_ref[...] = jnp.where(pl.program_id(0) == 0, part, prev + part)


def _dual_body(htr_ref, hre_ref, cbtr_ref, cbre_ref,
               qtr_ref, gtr_ref, itr_ref, sstr_ref,
               qre_ref, gre_ref, ire_ref, ssre_ref, *, rb, k):
    _vq_body(htr_ref, cbtr_ref, qtr_ref, gtr_ref, itr_ref, sstr_ref, rb=rb, k=k)
    _vq_body(hre_ref, cbre_ref, qre_ref, gre_ref, ire_ref, ssre_ref, rb=rb, k=k)


def kernel(h_tr, h_re, codebook_tr, codebook_re):
    b, d = h_tr.shape
    k = codebook_tr.shape[0]
    rb = min(256, b)
    nb = b // rb

    import functools
    body = functools.partial(_dual_body, rb=rb, k=k)

    row_spec = pl.BlockSpec((rb, d), lambda i: (i, 0))
    cb_spec = pl.BlockSpec((k, d), lambda i: (0, 0))
    q_spec = pl.BlockSpec((rb, k), lambda i: (i, 0))
    idx_spec = pl.BlockSpec((rb, 1), lambda i: (i, 0))
    ss_spec = pl.BlockSpec((1, 1), lambda i: (0, 0))

    f32 = jnp.float32
    out_shapes = (
        jax.ShapeDtypeStruct((b, k), f32),
        jax.ShapeDtypeStruct((b, d), f32),
        jax.ShapeDtypeStruct((b, 1), jnp.int32),
        jax.ShapeDtypeStruct((1, 1), f32),
        jax.ShapeDtypeStruct((b, k), f32),
        jax.ShapeDtypeStruct((b, d), f32),
        jax.ShapeDtypeStruct((b, 1), jnp.int32),
        jax.ShapeDtypeStruct((1, 1), f32),
    )
    out_specs = (q_spec, row_spec, idx_spec, ss_spec,
                 q_spec, row_spec, idx_spec, ss_spec)

    (q_tr, g_tr, i_tr, ss_tr, q_re, g_re, i_re, ss_re) = pl.pallas_call(
        body,
        grid=(nb,),
        in_specs=[row_spec, row_spec, cb_spec, cb_spec],
        out_specs=out_specs,
        out_shape=out_shapes,
        compiler_params=pltpu.CompilerParams(
            dimension_semantics=("arbitrary",)),
    )(h_tr, h_re, codebook_tr, codebook_re)

    n = jnp.float32(b * d)
    total_loss = (1.0 + _BETA) * (ss_tr[0, 0] / n) + \
                 (1.0 + _BETA) * (ss_re[0, 0] / n)
    idx_tr = i_tr.reshape(b)
    idx_re = i_re.reshape(b)
    return (q_tr, g_tr, g_tr, g_tr, idx_tr,
            q_re, g_re, g_re, g_re, idx_re,
            total_loss)
